# baseline (speedup 1.0000x reference)
import functools

import jax
import jax.numpy as jnp
from jax import lax
from jax.experimental import pallas as pl
from jax.experimental.pallas import tpu as pltpu
from jax.experimental.pallas import tpu_sc as plsc

_NC = 2
_NS = 16
_NW = _NC * _NS
_L = 16
_BB = 128


def _sc_gather(table, idx, embed_dim):
    batch, seq = idx.shape
    eh = embed_dim // 8

    mesh = plsc.VectorSubcoreMesh(core_axis_name="c", subcore_axis_name="s")

    @functools.partial(
        pl.kernel,
        out_type=jax.ShapeDtypeStruct((seq, eh, batch // _BB, 8, _BB), jnp.float32),
        mesh=mesh,
        scratch_types=[
            pltpu.VMEM((_BB, seq), jnp.int32),
            pltpu.VMEM((seq, _BB), jnp.int32),
            pltpu.VMEM((_BB, embed_dim), jnp.float32),
            pltpu.VMEM((_BB, embed_dim), jnp.float32),
            pltpu.VMEM((eh, 8, _BB), jnp.float32),
            pltpu.VMEM((eh, 8, _BB), jnp.float32),
            pltpu.SemaphoreType.DMA,
            pltpu.SemaphoreType.DMA,
            pltpu.SemaphoreType.DMA,
            pltpu.SemaphoreType.DMA,
        ],
        compiler_params=pltpu.CompilerParams(
            use_tc_tiling_on_sc=False, needs_layout_passes=False
        ),
    )
    def k(table_hbm, idx_hbm, out_hbm, idx_v, idxt_v, r0, r1, o0, o1,
          sg0, sg1, so0, so1):
        wid = lax.axis_index("s") * _NC + lax.axis_index("c")
        rows = (r0, r1)
        oblk = (o0, o1)
        semg = (sg0, sg1)
        semo = (so0, so1)

        bvec = [jnp.arange(_L, dtype=jnp.int32) + _L * h for h in range(_BB // _L)]

        pltpu.sync_copy(idx_hbm.at[pl.ds(wid * _BB, _BB)], idx_v)

        def tbody(t, carry):
            tvec = jnp.full((_L,), 0, jnp.int32) + t
            for h in range(_BB // _L):
                idxt_v[t, pl.ds(_L * h, _L)] = plsc.load_gather(
                    idx_v, [bvec[h], tvec]
                )
            return carry

        lax.fori_loop(0, seq, tbody, 0)

        def fire(t, buf, sem):
            pltpu.async_copy(table_hbm.at[idxt_v.at[t]], buf, sem)

        fire(0, r0, sg0)

        def step(t, carry):
            for b in range(2):

                @pl.when(t % 2 == b)
                def _():
                    rb, ob = rows[b], oblk[b]

                    @pl.when(t < seq - 1)
                    def _():
                        fire(t + 1, rows[1 - b], semg[1 - b])

                    pltpu.make_async_copy(
                        table_hbm.at[pl.ds(0, _BB)], rb, semg[b]
                    ).wait()

                    @pl.when(t >= 2)
                    def _():
                        pltpu.make_async_copy(
                            out_hbm.at[0, :, 0], ob, semo[b]
                        ).wait()

                    for e in range(embed_dim):
                        evec = jnp.full((_L,), e, jnp.int32)
                        for h in range(_BB // _L):
                            ob[e // 8, e % 8, pl.ds(_L * h, _L)] = (
                                plsc.load_gather(rb, [bvec[h], evec])
                            )

                    pltpu.async_copy(ob, out_hbm.at[t, :, wid], semo[b])

            return carry

        lax.fori_loop(0, seq, step, 0)

        pltpu.make_async_copy(out_hbm.at[0, :, 0], o0, so0).wait()
        pltpu.make_async_copy(out_hbm.at[0, :, 0], o1, so1).wait()

    return k(table, idx)


def kernel(inputs, embedding_matrix):
    batch, seq = inputs.shape
    vocab, embed_dim = embedding_matrix.shape
    out5 = _sc_gather(embedding_matrix, inputs.astype(jnp.int32), embed_dim)
    return out5.transpose(2, 4, 0, 1, 3).reshape(batch, seq, embed_dim)
